# baseline (speedup 1.0000x reference)
import functools

import jax
import jax.numpy as jnp
import numpy as np
from jax import lax
from jax.experimental import pallas as pl
from jax.experimental.pallas import tpu as pltpu
from jax.experimental.pallas import tpu_sc as plsc

_BLOSUM62 = np.array([
    [4, 0, -2, -1, -2, 0, -2, -1, -1, -1, -1, -2, -1, -1, -1, 1, 0, 0, -3, -2],
    [0, 9, -3, -4, -2, -3, -3, -1, -3, -1, -1, -3, -3, -3, -3, -1, -1, -1, -2, -2],
    [-2, -3, 6, 2, -3, -1, -1, -3, -1, -4, -3, 1, -1, 0, -2, 0, -1, -3, -4, -3],
    [-1, -4, 2, 5, -3, -2, 0, -3, 1, -3, -2, 0, -1, 2, 0, 0, -1, -2, -3, -2],
    [-2, -2, -3, -3, 6, -3, -1, 0, -3, 0, 0, -3, -4, -3, -3, -2, -2, -1, 1, 3],
    [0, -3, -1, -2, -3, 6, -2, -4, -2, -4, -3, 0, -2, -2, -2, 0, -2, -3, -2, -3],
    [-2, -3, -1, 0, -1, -2, 8, -3, -1, -3, -2, 1, -2, 0, 0, -1, -2, -3, -2, 2],
    [-1, -1, -3, -3, 0, -4, -3, 4, -3, 2, 1, -3, -3, -3, -3, -2, -1, 3, -3, -1],
    [-1, -3, -1, 1, -3, -2, -1, -3, 5, -2, -1, 0, -1, 1, 2, 0, -1, -2, -3, -2],
    [-1, -1, -4, -3, 0, -4, -3, 2, -2, 4, 2, -3, -3, -2, -2, -2, -1, 1, -2, -1],
    [-1, -1, -3, -2, 0, -3, -2, 1, -1, 2, 5, -2, -2, 0, -1, -1, -1, 1, -1, -1],
    [-2, -3, 1, 0, -3, 0, 1, -3, 0, -3, -2, 6, -2, 0, 0, 1, 0, -3, -4, -2],
    [-1, -3, -1, -1, -4, -2, -2, -3, -1, -3, -2, -2, 7, -1, -2, -1, -1, -2, -4, -3],
    [-1, -3, 0, 2, -3, -2, 0, -3, 1, -2, 0, 0, -1, 5, 1, 0, -1, -2, -2, -1],
    [-1, -3, -2, 0, -3, -2, 0, -3, 2, -2, -1, 0, -2, 1, 5, -1, -1, -3, -3, -2],
    [1, -1, 0, 0, -2, 0, -1, -2, 0, -2, -1, 1, -1, 0, -1, 4, 1, -2, -3, -2],
    [0, -1, -1, -1, -2, -2, -2, -1, -1, -1, -1, 0, -1, -1, -1, 1, 5, 0, -2, -2],
    [0, -1, -3, -2, -1, -3, -3, 3, -2, 1, 1, -3, -2, -2, -3, -2, 0, 4, -3, -1],
    [-3, -2, -4, -3, 1, -2, -2, -3, -3, -2, -1, -4, -4, -2, -3, -3, -2, -3, 11, 2],
    [-2, -2, -3, -2, 3, -3, 2, -1, -2, -1, -1, -2, -3, -1, -2, -2, -2, -1, 2, 7],
], dtype=np.float32)
_mean = _BLOSUM62.mean(axis=1, keepdims=True)
_std = _BLOSUM62.std(axis=1, keepdims=True) + 1e-08
_BLOSUM_PAD = np.zeros((32, 20), dtype=np.float32)
_BLOSUM_PAD[:20] = (_BLOSUM62 - _mean) / _std

_TABLE_ROWS = 32

_NC = 2
_NS = 16
_NW = _NC * _NS

_CHUNK = 128
_L = 16


def _fuse_table_body(bl_ref, w_ref, b_ref, out_ref):
    out_ref[...] = (
        jnp.dot(bl_ref[...], w_ref[...], preferred_element_type=jnp.float32)
        + b_ref[...]
    )


def _fused_table(W, b):
    d = W.shape[1]
    return pl.pallas_call(
        _fuse_table_body,
        out_shape=jax.ShapeDtypeStruct((_TABLE_ROWS, d), jnp.float32),
    )(jnp.asarray(_BLOSUM_PAD), W, b.reshape(1, d))


@functools.partial(jax.jit, static_argnames=("n", "d"))
def _sc_expand(table_flat, ids_flat, n, d):
    chunk = _CHUNK
    per_w = n // _NW
    n_chunks = per_w // chunk
    tbl_words = _TABLE_ROWS * d
    cd = chunk * d
    mesh = plsc.VectorSubcoreMesh(core_axis_name="c", subcore_axis_name="s")

    @functools.partial(
        pl.kernel,
        mesh=mesh,
        compiler_params=pltpu.CompilerParams(needs_layout_passes=False),
        out_type=jax.ShapeDtypeStruct((n * d,), jnp.float32),
        scratch_types=[
            pltpu.VMEM((tbl_words,), jnp.float32),
            pltpu.VMEM((2 * chunk,), jnp.int32),
            pltpu.VMEM((2 * cd,), jnp.float32),
            pltpu.SemaphoreType.DMA,
            pltpu.SemaphoreType.DMA,
            pltpu.SemaphoreType.DMA,
            pltpu.SemaphoreType.DMA,
        ],
    )
    def k(table_hbm, ids_hbm, out_hbm, tbl_v, ids_v, obuf_v, si0, si1, so0, so1):
        wid = lax.axis_index("s") * _NC + lax.axis_index("c")
        tok0 = wid * per_w
        pltpu.sync_copy(table_hbm, tbl_v)
        pltpu.async_copy(ids_hbm.at[pl.ds(tok0, chunk)],
                         ids_v.at[pl.ds(0, chunk)], si0)
        pltpu.async_copy(ids_hbm.at[pl.ds(tok0 + chunk, chunk)],
                         ids_v.at[pl.ds(chunk, chunk)], si1)

        lane = lax.iota(jnp.int32, _L)
        wout0 = lane * d

        def step(c, carry):
            p = lax.rem(c, 2)
            ioff = p * chunk
            ooff = p * cd

            @pl.when(p == 0)
            def _():
                pltpu.make_async_copy(ids_hbm.at[pl.ds(tok0, chunk)],
                                      ids_v.at[pl.ds(0, chunk)], si0).wait()

            @pl.when(p == 1)
            def _():
                pltpu.make_async_copy(ids_hbm.at[pl.ds(tok0, chunk)],
                                      ids_v.at[pl.ds(0, chunk)], si1).wait()

            @pl.when(jnp.logical_and(c >= 2, p == 0))
            def _():
                pltpu.make_async_copy(obuf_v.at[pl.ds(0, cd)],
                                      out_hbm.at[pl.ds(0, cd)], so0).wait()

            @pl.when(jnp.logical_and(c >= 2, p == 1))
            def _():
                pltpu.make_async_copy(obuf_v.at[pl.ds(0, cd)],
                                      out_hbm.at[pl.ds(0, cd)], so1).wait()

            for t0 in range(0, chunk, _L):
                ids16 = ids_v[pl.ds(ioff + t0, _L)]
                addr = ids16 * d
                wbase = wout0 + (ooff + t0 * d)
                for j in range(d):
                    v = plsc.load_gather(tbl_v, [addr + j])
                    plsc.store_scatter(obuf_v, [wbase + j], v)

            dst = out_hbm.at[pl.ds((tok0 + c * chunk) * d, cd)]

            @pl.when(p == 0)
            def _():
                pltpu.async_copy(obuf_v.at[pl.ds(0, cd)], dst, so0)

            @pl.when(p == 1)
            def _():
                pltpu.async_copy(obuf_v.at[pl.ds(cd, cd)], dst, so1)

            @pl.when(jnp.logical_and(c + 2 < n_chunks, p == 0))
            def _():
                pltpu.async_copy(
                    ids_hbm.at[pl.ds(tok0 + (c + 2) * chunk, chunk)],
                    ids_v.at[pl.ds(0, chunk)], si0)

            @pl.when(jnp.logical_and(c + 2 < n_chunks, p == 1))
            def _():
                pltpu.async_copy(
                    ids_hbm.at[pl.ds(tok0 + (c + 2) * chunk, chunk)],
                    ids_v.at[pl.ds(chunk, chunk)], si1)

            return carry

        lax.fori_loop(0, n_chunks, step, 0)

        pltpu.make_async_copy(obuf_v.at[pl.ds(0, cd)],
                              out_hbm.at[pl.ds(0, cd)], so0).wait()
        pltpu.make_async_copy(obuf_v.at[pl.ds(0, cd)],
                              out_hbm.at[pl.ds(0, cd)], so1).wait()

    return k(table_flat, ids_flat)


def kernel(token_ids, W, b):
    bsz, seq = token_ids.shape
    d = W.shape[1]
    n = bsz * seq
    table = _fused_table(W, b)
    out = _sc_expand(table.reshape(_TABLE_ROWS * d), token_ids.reshape(n), n, d)
    return out.reshape(bsz, seq, d)


# speedup vs baseline: 2.5354x; 2.5354x over previous
import functools

import jax
import jax.numpy as jnp
import numpy as np
from jax import lax
from jax.experimental import pallas as pl
from jax.experimental.pallas import tpu as pltpu
from jax.experimental.pallas import tpu_sc as plsc

_BLOSUM62 = np.array([
    [4, 0, -2, -1, -2, 0, -2, -1, -1, -1, -1, -2, -1, -1, -1, 1, 0, 0, -3, -2],
    [0, 9, -3, -4, -2, -3, -3, -1, -3, -1, -1, -3, -3, -3, -3, -1, -1, -1, -2, -2],
    [-2, -3, 6, 2, -3, -1, -1, -3, -1, -4, -3, 1, -1, 0, -2, 0, -1, -3, -4, -3],
    [-1, -4, 2, 5, -3, -2, 0, -3, 1, -3, -2, 0, -1, 2, 0, 0, -1, -2, -3, -2],
    [-2, -2, -3, -3, 6, -3, -1, 0, -3, 0, 0, -3, -4, -3, -3, -2, -2, -1, 1, 3],
    [0, -3, -1, -2, -3, 6, -2, -4, -2, -4, -3, 0, -2, -2, -2, 0, -2, -3, -2, -3],
    [-2, -3, -1, 0, -1, -2, 8, -3, -1, -3, -2, 1, -2, 0, 0, -1, -2, -3, -2, 2],
    [-1, -1, -3, -3, 0, -4, -3, 4, -3, 2, 1, -3, -3, -3, -3, -2, -1, 3, -3, -1],
    [-1, -3, -1, 1, -3, -2, -1, -3, 5, -2, -1, 0, -1, 1, 2, 0, -1, -2, -3, -2],
    [-1, -1, -4, -3, 0, -4, -3, 2, -2, 4, 2, -3, -3, -2, -2, -2, -1, 1, -2, -1],
    [-1, -1, -3, -2, 0, -3, -2, 1, -1, 2, 5, -2, -2, 0, -1, -1, -1, 1, -1, -1],
    [-2, -3, 1, 0, -3, 0, 1, -3, 0, -3, -2, 6, -2, 0, 0, 1, 0, -3, -4, -2],
    [-1, -3, -1, -1, -4, -2, -2, -3, -1, -3, -2, -2, 7, -1, -2, -1, -1, -2, -4, -3],
    [-1, -3, 0, 2, -3, -2, 0, -3, 1, -2, 0, 0, -1, 5, 1, 0, -1, -2, -2, -1],
    [-1, -3, -2, 0, -3, -2, 0, -3, 2, -2, -1, 0, -2, 1, 5, -1, -1, -3, -3, -2],
    [1, -1, 0, 0, -2, 0, -1, -2, 0, -2, -1, 1, -1, 0, -1, 4, 1, -2, -3, -2],
    [0, -1, -1, -1, -2, -2, -2, -1, -1, -1, -1, 0, -1, -1, -1, 1, 5, 0, -2, -2],
    [0, -1, -3, -2, -1, -3, -3, 3, -2, 1, 1, -3, -2, -2, -3, -2, 0, 4, -3, -1],
    [-3, -2, -4, -3, 1, -2, -2, -3, -3, -2, -1, -4, -4, -2, -3, -3, -2, -3, 11, 2],
    [-2, -2, -3, -2, 3, -3, 2, -1, -2, -1, -1, -2, -3, -1, -2, -2, -2, -1, 2, 7],
], dtype=np.float32)
_mean = _BLOSUM62.mean(axis=1, keepdims=True)
_std = _BLOSUM62.std(axis=1, keepdims=True) + 1e-08
_BLOSUM_PAD = np.zeros((32, 20), dtype=np.float32)
_BLOSUM_PAD[:20] = (_BLOSUM62 - _mean) / _std

_TABLE_ROWS = 32

_NC = 2
_NS = 16
_NW = _NC * _NS

_CHUNK = 128
_L = 16


def _fuse_table_body(bl_ref, w_ref, b_ref, out_ref):
    out_ref[...] = (
        jnp.dot(bl_ref[...], w_ref[...], preferred_element_type=jnp.float32)
        + b_ref[...]
    )


def _fused_table(W, b):
    d = W.shape[1]
    return pl.pallas_call(
        _fuse_table_body,
        out_shape=jax.ShapeDtypeStruct((_TABLE_ROWS, d), jnp.float32),
    )(jnp.asarray(_BLOSUM_PAD), W, b.reshape(1, d))


@functools.partial(jax.jit, static_argnames=("n", "d"))
def _sc_expand(table_flat, ids_flat, n, d):
    chunk = _CHUNK
    per_w = n // _NW
    n_chunks = per_w // chunk
    tbl_words = _TABLE_ROWS * d
    cd = chunk * d
    mesh = plsc.VectorSubcoreMesh(core_axis_name="c", subcore_axis_name="s")

    @functools.partial(
        pl.kernel,
        mesh=mesh,
        compiler_params=pltpu.CompilerParams(needs_layout_passes=False),
        out_type=jax.ShapeDtypeStruct((n * d,), jnp.float32),
        scratch_types=[
            pltpu.VMEM((tbl_words,), jnp.float32),
            pltpu.VMEM((2 * chunk,), jnp.int32),
            pltpu.VMEM((2 * cd,), jnp.float32),
            pltpu.SemaphoreType.DMA,
            pltpu.SemaphoreType.DMA,
            pltpu.SemaphoreType.DMA,
            pltpu.SemaphoreType.DMA,
        ],
    )
    def k(table_hbm, ids_hbm, out_hbm, tbl_v, ids_v, obuf_v, si0, si1, so0, so1):
        wid = lax.axis_index("s") * _NC + lax.axis_index("c")
        tok0 = wid * per_w
        pltpu.sync_copy(table_hbm, tbl_v)
        pltpu.async_copy(ids_hbm.at[pl.ds(tok0, chunk)],
                         ids_v.at[pl.ds(0, chunk)], si0)
        pltpu.async_copy(ids_hbm.at[pl.ds(tok0 + chunk, chunk)],
                         ids_v.at[pl.ds(chunk, chunk)], si1)

        lane = lax.iota(jnp.int32, _L)
        wout0 = lane * d

        def step(c, carry):
            p = lax.rem(c, 2)
            ioff = p * chunk
            ooff = p * cd

            @pl.when(p == 0)
            def _():
                pltpu.make_async_copy(ids_hbm.at[pl.ds(tok0, chunk)],
                                      ids_v.at[pl.ds(0, chunk)], si0).wait()

            @pl.when(p == 1)
            def _():
                pltpu.make_async_copy(ids_hbm.at[pl.ds(tok0, chunk)],
                                      ids_v.at[pl.ds(0, chunk)], si1).wait()

            @pl.when(jnp.logical_and(c >= 2, p == 0))
            def _():
                pltpu.make_async_copy(obuf_v.at[pl.ds(0, cd)],
                                      out_hbm.at[pl.ds(0, cd)], so0).wait()

            @pl.when(jnp.logical_and(c >= 2, p == 1))
            def _():
                pltpu.make_async_copy(obuf_v.at[pl.ds(0, cd)],
                                      out_hbm.at[pl.ds(0, cd)], so1).wait()

            for t0 in range(0, chunk, _L):
                ids16 = ids_v[pl.ds(ioff + t0, _L)]
                addr = ids16 * d
                wbase = wout0 + (ooff + t0 * d)

                @plsc.parallel_loop(0, d, unroll=16)
                def _(j):
                    v = plsc.load_gather(tbl_v, [addr + j])
                    plsc.store_scatter(obuf_v, [wbase + j], v)

            dst = out_hbm.at[pl.ds((tok0 + c * chunk) * d, cd)]

            @pl.when(p == 0)
            def _():
                pltpu.async_copy(obuf_v.at[pl.ds(0, cd)], dst, so0)

            @pl.when(p == 1)
            def _():
                pltpu.async_copy(obuf_v.at[pl.ds(cd, cd)], dst, so1)

            @pl.when(jnp.logical_and(c + 2 < n_chunks, p == 0))
            def _():
                pltpu.async_copy(
                    ids_hbm.at[pl.ds(tok0 + (c + 2) * chunk, chunk)],
                    ids_v.at[pl.ds(0, chunk)], si0)

            @pl.when(jnp.logical_and(c + 2 < n_chunks, p == 1))
            def _():
                pltpu.async_copy(
                    ids_hbm.at[pl.ds(tok0 + (c + 2) * chunk, chunk)],
                    ids_v.at[pl.ds(chunk, chunk)], si1)

            return carry

        lax.fori_loop(0, n_chunks, step, 0)

        pltpu.make_async_copy(obuf_v.at[pl.ds(0, cd)],
                              out_hbm.at[pl.ds(0, cd)], so0).wait()
        pltpu.make_async_copy(obuf_v.at[pl.ds(0, cd)],
                              out_hbm.at[pl.ds(0, cd)], so1).wait()

    return k(table_flat, ids_flat)


def kernel(token_ids, W, b):
    bsz, seq = token_ids.shape
    d = W.shape[1]
    n = bsz * seq
    table = _fused_table(W, b)
    out = _sc_expand(table.reshape(_TABLE_ROWS * d), token_ids.reshape(n), n, d)
    return out.reshape(bsz, seq, d)


# speedup vs baseline: 17.5697x; 6.9297x over previous
import functools

import jax
import jax.numpy as jnp
import numpy as np
from jax import lax
from jax.experimental import pallas as pl
from jax.experimental.pallas import tpu as pltpu
from jax.experimental.pallas import tpu_sc as plsc

_BLOSUM62 = np.array([
    [4, 0, -2, -1, -2, 0, -2, -1, -1, -1, -1, -2, -1, -1, -1, 1, 0, 0, -3, -2],
    [0, 9, -3, -4, -2, -3, -3, -1, -3, -1, -1, -3, -3, -3, -3, -1, -1, -1, -2, -2],
    [-2, -3, 6, 2, -3, -1, -1, -3, -1, -4, -3, 1, -1, 0, -2, 0, -1, -3, -4, -3],
    [-1, -4, 2, 5, -3, -2, 0, -3, 1, -3, -2, 0, -1, 2, 0, 0, -1, -2, -3, -2],
    [-2, -2, -3, -3, 6, -3, -1, 0, -3, 0, 0, -3, -4, -3, -3, -2, -2, -1, 1, 3],
    [0, -3, -1, -2, -3, 6, -2, -4, -2, -4, -3, 0, -2, -2, -2, 0, -2, -3, -2, -3],
    [-2, -3, -1, 0, -1, -2, 8, -3, -1, -3, -2, 1, -2, 0, 0, -1, -2, -3, -2, 2],
    [-1, -1, -3, -3, 0, -4, -3, 4, -3, 2, 1, -3, -3, -3, -3, -2, -1, 3, -3, -1],
    [-1, -3, -1, 1, -3, -2, -1, -3, 5, -2, -1, 0, -1, 1, 2, 0, -1, -2, -3, -2],
    [-1, -1, -4, -3, 0, -4, -3, 2, -2, 4, 2, -3, -3, -2, -2, -2, -1, 1, -2, -1],
    [-1, -1, -3, -2, 0, -3, -2, 1, -1, 2, 5, -2, -2, 0, -1, -1, -1, 1, -1, -1],
    [-2, -3, 1, 0, -3, 0, 1, -3, 0, -3, -2, 6, -2, 0, 0, 1, 0, -3, -4, -2],
    [-1, -3, -1, -1, -4, -2, -2, -3, -1, -3, -2, -2, 7, -1, -2, -1, -1, -2, -4, -3],
    [-1, -3, 0, 2, -3, -2, 0, -3, 1, -2, 0, 0, -1, 5, 1, 0, -1, -2, -2, -1],
    [-1, -3, -2, 0, -3, -2, 0, -3, 2, -2, -1, 0, -2, 1, 5, -1, -1, -3, -3, -2],
    [1, -1, 0, 0, -2, 0, -1, -2, 0, -2, -1, 1, -1, 0, -1, 4, 1, -2, -3, -2],
    [0, -1, -1, -1, -2, -2, -2, -1, -1, -1, -1, 0, -1, -1, -1, 1, 5, 0, -2, -2],
    [0, -1, -3, -2, -1, -3, -3, 3, -2, 1, 1, -3, -2, -2, -3, -2, 0, 4, -3, -1],
    [-3, -2, -4, -3, 1, -2, -2, -3, -3, -2, -1, -4, -4, -2, -3, -3, -2, -3, 11, 2],
    [-2, -2, -3, -2, 3, -3, 2, -1, -2, -1, -1, -2, -3, -1, -2, -2, -2, -1, 2, 7],
], dtype=np.float32)
_mean = _BLOSUM62.mean(axis=1, keepdims=True)
_std = _BLOSUM62.std(axis=1, keepdims=True) + 1e-08
_BLOSUM_PAD = np.zeros((32, 20), dtype=np.float32)
_BLOSUM_PAD[:20] = (_BLOSUM62 - _mean) / _std

_TABLE_ROWS = 32

_NC = 2
_NS = 16
_NW = _NC * _NS

_CHUNK = 128
_L = 16


def _fuse_table_body(bl_ref, w_ref, b_ref, out_ref):
    out_ref[...] = (
        jnp.dot(bl_ref[...], w_ref[...], preferred_element_type=jnp.float32)
        + b_ref[...]
    )


def _fused_table(W, b):
    d = W.shape[1]
    return pl.pallas_call(
        _fuse_table_body,
        out_shape=jax.ShapeDtypeStruct((_TABLE_ROWS, d), jnp.float32),
    )(jnp.asarray(_BLOSUM_PAD), W, b.reshape(1, d))


@functools.partial(jax.jit, static_argnames=("n", "d"))
def _sc_expand(table_flat, ids_flat, n, d):
    chunk = _CHUNK
    per_w = n // _NW
    n_chunks = per_w // chunk
    tbl_words = _TABLE_ROWS * d
    cd = chunk * d
    mesh = plsc.VectorSubcoreMesh(core_axis_name="c", subcore_axis_name="s")

    @functools.partial(
        pl.kernel,
        mesh=mesh,
        compiler_params=pltpu.CompilerParams(needs_layout_passes=False),
        out_type=jax.ShapeDtypeStruct((n * d,), jnp.float32),
        scratch_types=[
            pltpu.VMEM((tbl_words,), jnp.float32),
            pltpu.VMEM((2 * chunk,), jnp.int32),
            pltpu.VMEM((2 * cd,), jnp.float32),
            pltpu.SemaphoreType.DMA,
            pltpu.SemaphoreType.DMA,
            pltpu.SemaphoreType.DMA,
            pltpu.SemaphoreType.DMA,
        ],
    )
    def k(table_hbm, ids_hbm, out_hbm, tbl_v, ids_v, obuf_v, si0, si1, so0, so1):
        wid = lax.axis_index("s") * _NC + lax.axis_index("c")
        tok0 = wid * per_w
        pltpu.sync_copy(table_hbm, tbl_v)
        pltpu.async_copy(ids_hbm.at[pl.ds(tok0, chunk)],
                         ids_v.at[pl.ds(0, chunk)], si0)
        pltpu.async_copy(ids_hbm.at[pl.ds(tok0 + chunk, chunk)],
                         ids_v.at[pl.ds(chunk, chunk)], si1)

        lane = lax.iota(jnp.int32, _L)
        wout0 = lane * d

        def step(c, carry):
            p = lax.rem(c, 2)
            ioff = p * chunk
            ooff = p * cd

            @pl.when(p == 0)
            def _():
                pltpu.make_async_copy(ids_hbm.at[pl.ds(tok0, chunk)],
                                      ids_v.at[pl.ds(0, chunk)], si0).wait()

            @pl.when(p == 1)
            def _():
                pltpu.make_async_copy(ids_hbm.at[pl.ds(tok0, chunk)],
                                      ids_v.at[pl.ds(0, chunk)], si1).wait()

            @pl.when(jnp.logical_and(c >= 2, p == 0))
            def _():
                pltpu.make_async_copy(obuf_v.at[pl.ds(0, cd)],
                                      out_hbm.at[pl.ds(0, cd)], so0).wait()

            @pl.when(jnp.logical_and(c >= 2, p == 1))
            def _():
                pltpu.make_async_copy(obuf_v.at[pl.ds(0, cd)],
                                      out_hbm.at[pl.ds(0, cd)], so1).wait()

            @plsc.parallel_loop(0, chunk // _L, unroll=1)
            def _(blk):
                ids16 = ids_v[pl.ds(ioff + blk * _L, _L)]
                obase0 = ooff + blk * (_L * d)
                for l in range(_L):
                    rbase = ids16[l] * d
                    obase = obase0 + l * d
                    for jj in range(0, d, _L):
                        obuf_v[pl.ds(obase + jj, _L)] = (
                            tbl_v[pl.ds(rbase + jj, _L)])

            dst = out_hbm.at[pl.ds((tok0 + c * chunk) * d, cd)]

            @pl.when(p == 0)
            def _():
                pltpu.async_copy(obuf_v.at[pl.ds(0, cd)], dst, so0)

            @pl.when(p == 1)
            def _():
                pltpu.async_copy(obuf_v.at[pl.ds(cd, cd)], dst, so1)

            @pl.when(jnp.logical_and(c + 2 < n_chunks, p == 0))
            def _():
                pltpu.async_copy(
                    ids_hbm.at[pl.ds(tok0 + (c + 2) * chunk, chunk)],
                    ids_v.at[pl.ds(0, chunk)], si0)

            @pl.when(jnp.logical_and(c + 2 < n_chunks, p == 1))
            def _():
                pltpu.async_copy(
                    ids_hbm.at[pl.ds(tok0 + (c + 2) * chunk, chunk)],
                    ids_v.at[pl.ds(chunk, chunk)], si1)

            return carry

        lax.fori_loop(0, n_chunks, step, 0)

        pltpu.make_async_copy(obuf_v.at[pl.ds(0, cd)],
                              out_hbm.at[pl.ds(0, cd)], so0).wait()
        pltpu.make_async_copy(obuf_v.at[pl.ds(0, cd)],
                              out_hbm.at[pl.ds(0, cd)], so1).wait()

    return k(table_flat, ids_flat)


def kernel(token_ids, W, b):
    bsz, seq = token_ids.shape
    d = W.shape[1]
    n = bsz * seq
    table = _fused_table(W, b)
    out = _sc_expand(table.reshape(_TABLE_ROWS * d), token_ids.reshape(n), n, d)
    return out.reshape(bsz, seq, d)


# speedup vs baseline: 20.5238x; 1.1681x over previous
import functools

import jax
import jax.numpy as jnp
import numpy as np
from jax import lax
from jax.experimental import pallas as pl
from jax.experimental.pallas import tpu as pltpu
from jax.experimental.pallas import tpu_sc as plsc

_BLOSUM62 = np.array([
    [4, 0, -2, -1, -2, 0, -2, -1, -1, -1, -1, -2, -1, -1, -1, 1, 0, 0, -3, -2],
    [0, 9, -3, -4, -2, -3, -3, -1, -3, -1, -1, -3, -3, -3, -3, -1, -1, -1, -2, -2],
    [-2, -3, 6, 2, -3, -1, -1, -3, -1, -4, -3, 1, -1, 0, -2, 0, -1, -3, -4, -3],
    [-1, -4, 2, 5, -3, -2, 0, -3, 1, -3, -2, 0, -1, 2, 0, 0, -1, -2, -3, -2],
    [-2, -2, -3, -3, 6, -3, -1, 0, -3, 0, 0, -3, -4, -3, -3, -2, -2, -1, 1, 3],
    [0, -3, -1, -2, -3, 6, -2, -4, -2, -4, -3, 0, -2, -2, -2, 0, -2, -3, -2, -3],
    [-2, -3, -1, 0, -1, -2, 8, -3, -1, -3, -2, 1, -2, 0, 0, -1, -2, -3, -2, 2],
    [-1, -1, -3, -3, 0, -4, -3, 4, -3, 2, 1, -3, -3, -3, -3, -2, -1, 3, -3, -1],
    [-1, -3, -1, 1, -3, -2, -1, -3, 5, -2, -1, 0, -1, 1, 2, 0, -1, -2, -3, -2],
    [-1, -1, -4, -3, 0, -4, -3, 2, -2, 4, 2, -3, -3, -2, -2, -2, -1, 1, -2, -1],
    [-1, -1, -3, -2, 0, -3, -2, 1, -1, 2, 5, -2, -2, 0, -1, -1, -1, 1, -1, -1],
    [-2, -3, 1, 0, -3, 0, 1, -3, 0, -3, -2, 6, -2, 0, 0, 1, 0, -3, -4, -2],
    [-1, -3, -1, -1, -4, -2, -2, -3, -1, -3, -2, -2, 7, -1, -2, -1, -1, -2, -4, -3],
    [-1, -3, 0, 2, -3, -2, 0, -3, 1, -2, 0, 0, -1, 5, 1, 0, -1, -2, -2, -1],
    [-1, -3, -2, 0, -3, -2, 0, -3, 2, -2, -1, 0, -2, 1, 5, -1, -1, -3, -3, -2],
    [1, -1, 0, 0, -2, 0, -1, -2, 0, -2, -1, 1, -1, 0, -1, 4, 1, -2, -3, -2],
    [0, -1, -1, -1, -2, -2, -2, -1, -1, -1, -1, 0, -1, -1, -1, 1, 5, 0, -2, -2],
    [0, -1, -3, -2, -1, -3, -3, 3, -2, 1, 1, -3, -2, -2, -3, -2, 0, 4, -3, -1],
    [-3, -2, -4, -3, 1, -2, -2, -3, -3, -2, -1, -4, -4, -2, -3, -3, -2, -3, 11, 2],
    [-2, -2, -3, -2, 3, -3, 2, -1, -2, -1, -1, -2, -3, -1, -2, -2, -2, -1, 2, 7],
], dtype=np.float32)
_mean = _BLOSUM62.mean(axis=1, keepdims=True)
_std = _BLOSUM62.std(axis=1, keepdims=True) + 1e-08
_BLOSUM_PAD = np.zeros((32, 20), dtype=np.float32)
_BLOSUM_PAD[:20] = (_BLOSUM62 - _mean) / _std

_TABLE_ROWS = 32

_NC = 2
_NS = 16
_NW = _NC * _NS

_CHUNK = 256
_L = 16


def _fuse_table_body(bl_ref, w_ref, b_ref, out_ref):
    out_ref[...] = (
        jnp.dot(bl_ref[...], w_ref[...], preferred_element_type=jnp.float32)
        + b_ref[...]
    )


def _fused_table(W, b):
    d = W.shape[1]
    return pl.pallas_call(
        _fuse_table_body,
        out_shape=jax.ShapeDtypeStruct((_TABLE_ROWS, d), jnp.float32),
    )(jnp.asarray(_BLOSUM_PAD), W, b.reshape(1, d))


@functools.partial(jax.jit, static_argnames=("n", "d"))
def _sc_expand(table_flat, ids_flat, n, d):
    chunk = _CHUNK
    per_w = n // _NW
    n_chunks = per_w // chunk
    tbl_words = _TABLE_ROWS * d
    cd = chunk * d
    mesh = plsc.VectorSubcoreMesh(core_axis_name="c", subcore_axis_name="s")

    @functools.partial(
        pl.kernel,
        mesh=mesh,
        compiler_params=pltpu.CompilerParams(needs_layout_passes=False),
        out_type=jax.ShapeDtypeStruct((n * d,), jnp.float32),
        scratch_types=[
            pltpu.VMEM((tbl_words,), jnp.float32),
            pltpu.VMEM((2 * chunk,), jnp.int32),
            pltpu.VMEM((2 * cd,), jnp.float32),
            pltpu.SemaphoreType.DMA,
            pltpu.SemaphoreType.DMA,
            pltpu.SemaphoreType.DMA,
            pltpu.SemaphoreType.DMA,
        ],
    )
    def k(table_hbm, ids_hbm, out_hbm, tbl_v, ids_v, obuf_v, si0, si1, so0, so1):
        wid = lax.axis_index("s") * _NC + lax.axis_index("c")
        tok0 = wid * per_w
        pltpu.sync_copy(table_hbm, tbl_v)
        pltpu.async_copy(ids_hbm.at[pl.ds(tok0, chunk)],
                         ids_v.at[pl.ds(0, chunk)], si0)
        pltpu.async_copy(ids_hbm.at[pl.ds(tok0 + chunk, chunk)],
                         ids_v.at[pl.ds(chunk, chunk)], si1)

        lane = lax.iota(jnp.int32, _L)
        wout0 = lane * d

        def step(c, carry):
            p = lax.rem(c, 2)
            ioff = p * chunk
            ooff = p * cd

            @pl.when(p == 0)
            def _():
                pltpu.make_async_copy(ids_hbm.at[pl.ds(tok0, chunk)],
                                      ids_v.at[pl.ds(0, chunk)], si0).wait()

            @pl.when(p == 1)
            def _():
                pltpu.make_async_copy(ids_hbm.at[pl.ds(tok0, chunk)],
                                      ids_v.at[pl.ds(0, chunk)], si1).wait()

            @pl.when(jnp.logical_and(c >= 2, p == 0))
            def _():
                pltpu.make_async_copy(obuf_v.at[pl.ds(0, cd)],
                                      out_hbm.at[pl.ds(0, cd)], so0).wait()

            @pl.when(jnp.logical_and(c >= 2, p == 1))
            def _():
                pltpu.make_async_copy(obuf_v.at[pl.ds(0, cd)],
                                      out_hbm.at[pl.ds(0, cd)], so1).wait()

            @plsc.parallel_loop(0, chunk // _L, unroll=1)
            def _(blk):
                ids16 = ids_v[pl.ds(ioff + blk * _L, _L)]
                obase0 = ooff + blk * (_L * d)
                for l in range(_L):
                    rbase = ids16[l] * d
                    obase = obase0 + l * d
                    for jj in range(0, d, _L):
                        obuf_v[pl.ds(obase + jj, _L)] = (
                            tbl_v[pl.ds(rbase + jj, _L)])

            dst = out_hbm.at[pl.ds((tok0 + c * chunk) * d, cd)]

            @pl.when(p == 0)
            def _():
                pltpu.async_copy(obuf_v.at[pl.ds(0, cd)], dst, so0)

            @pl.when(p == 1)
            def _():
                pltpu.async_copy(obuf_v.at[pl.ds(cd, cd)], dst, so1)

            @pl.when(jnp.logical_and(c + 2 < n_chunks, p == 0))
            def _():
                pltpu.async_copy(
                    ids_hbm.at[pl.ds(tok0 + (c + 2) * chunk, chunk)],
                    ids_v.at[pl.ds(0, chunk)], si0)

            @pl.when(jnp.logical_and(c + 2 < n_chunks, p == 1))
            def _():
                pltpu.async_copy(
                    ids_hbm.at[pl.ds(tok0 + (c + 2) * chunk, chunk)],
                    ids_v.at[pl.ds(chunk, chunk)], si1)

            return carry

        lax.fori_loop(0, n_chunks, step, 0)

        pltpu.make_async_copy(obuf_v.at[pl.ds(0, cd)],
                              out_hbm.at[pl.ds(0, cd)], so0).wait()
        pltpu.make_async_copy(obuf_v.at[pl.ds(0, cd)],
                              out_hbm.at[pl.ds(0, cd)], so1).wait()

    return k(table_flat, ids_flat)


def kernel(token_ids, W, b):
    bsz, seq = token_ids.shape
    d = W.shape[1]
    n = bsz * seq
    table = _fused_table(W, b)
    out = _sc_expand(table.reshape(_TABLE_ROWS * d), token_ids.reshape(n), n, d)
    return out.reshape(bsz, seq, d)


# speedup vs baseline: 22.3157x; 1.0873x over previous
import functools

import jax
import jax.numpy as jnp
import numpy as np
from jax import lax
from jax.experimental import pallas as pl
from jax.experimental.pallas import tpu as pltpu
from jax.experimental.pallas import tpu_sc as plsc

_BLOSUM62 = np.array([
    [4, 0, -2, -1, -2, 0, -2, -1, -1, -1, -1, -2, -1, -1, -1, 1, 0, 0, -3, -2],
    [0, 9, -3, -4, -2, -3, -3, -1, -3, -1, -1, -3, -3, -3, -3, -1, -1, -1, -2, -2],
    [-2, -3, 6, 2, -3, -1, -1, -3, -1, -4, -3, 1, -1, 0, -2, 0, -1, -3, -4, -3],
    [-1, -4, 2, 5, -3, -2, 0, -3, 1, -3, -2, 0, -1, 2, 0, 0, -1, -2, -3, -2],
    [-2, -2, -3, -3, 6, -3, -1, 0, -3, 0, 0, -3, -4, -3, -3, -2, -2, -1, 1, 3],
    [0, -3, -1, -2, -3, 6, -2, -4, -2, -4, -3, 0, -2, -2, -2, 0, -2, -3, -2, -3],
    [-2, -3, -1, 0, -1, -2, 8, -3, -1, -3, -2, 1, -2, 0, 0, -1, -2, -3, -2, 2],
    [-1, -1, -3, -3, 0, -4, -3, 4, -3, 2, 1, -3, -3, -3, -3, -2, -1, 3, -3, -1],
    [-1, -3, -1, 1, -3, -2, -1, -3, 5, -2, -1, 0, -1, 1, 2, 0, -1, -2, -3, -2],
    [-1, -1, -4, -3, 0, -4, -3, 2, -2, 4, 2, -3, -3, -2, -2, -2, -1, 1, -2, -1],
    [-1, -1, -3, -2, 0, -3, -2, 1, -1, 2, 5, -2, -2, 0, -1, -1, -1, 1, -1, -1],
    [-2, -3, 1, 0, -3, 0, 1, -3, 0, -3, -2, 6, -2, 0, 0, 1, 0, -3, -4, -2],
    [-1, -3, -1, -1, -4, -2, -2, -3, -1, -3, -2, -2, 7, -1, -2, -1, -1, -2, -4, -3],
    [-1, -3, 0, 2, -3, -2, 0, -3, 1, -2, 0, 0, -1, 5, 1, 0, -1, -2, -2, -1],
    [-1, -3, -2, 0, -3, -2, 0, -3, 2, -2, -1, 0, -2, 1, 5, -1, -1, -3, -3, -2],
    [1, -1, 0, 0, -2, 0, -1, -2, 0, -2, -1, 1, -1, 0, -1, 4, 1, -2, -3, -2],
    [0, -1, -1, -1, -2, -2, -2, -1, -1, -1, -1, 0, -1, -1, -1, 1, 5, 0, -2, -2],
    [0, -1, -3, -2, -1, -3, -3, 3, -2, 1, 1, -3, -2, -2, -3, -2, 0, 4, -3, -1],
    [-3, -2, -4, -3, 1, -2, -2, -3, -3, -2, -1, -4, -4, -2, -3, -3, -2, -3, 11, 2],
    [-2, -2, -3, -2, 3, -3, 2, -1, -2, -1, -1, -2, -3, -1, -2, -2, -2, -1, 2, 7],
], dtype=np.float32)
_mean = _BLOSUM62.mean(axis=1, keepdims=True)
_std = _BLOSUM62.std(axis=1, keepdims=True) + 1e-08
_BLOSUM_PAD = np.zeros((32, 20), dtype=np.float32)
_BLOSUM_PAD[:20] = (_BLOSUM62 - _mean) / _std

_TABLE_ROWS = 32

_NC = 2
_NS = 16
_NW = _NC * _NS

_CHUNK = 320
_L = 16


def _fuse_table_body(bl_ref, w_ref, b_ref, out_ref):
    out_ref[...] = (
        jnp.dot(bl_ref[...], w_ref[...], preferred_element_type=jnp.float32)
        + b_ref[...]
    )


def _fused_table(W, b):
    d = W.shape[1]
    return pl.pallas_call(
        _fuse_table_body,
        out_shape=jax.ShapeDtypeStruct((_TABLE_ROWS, d), jnp.float32),
    )(jnp.asarray(_BLOSUM_PAD), W, b.reshape(1, d))


@functools.partial(jax.jit, static_argnames=("n", "d"))
def _sc_expand(table_flat, ids_flat, n, d):
    chunk = _CHUNK
    per_w = n // _NW
    n_chunks = per_w // chunk
    tbl_words = _TABLE_ROWS * d
    cd = chunk * d
    mesh = plsc.VectorSubcoreMesh(core_axis_name="c", subcore_axis_name="s")

    @functools.partial(
        pl.kernel,
        mesh=mesh,
        compiler_params=pltpu.CompilerParams(needs_layout_passes=False),
        out_type=jax.ShapeDtypeStruct((n * d,), jnp.float32),
        scratch_types=[
            pltpu.VMEM((tbl_words,), jnp.float32),
            pltpu.VMEM((2 * chunk,), jnp.int32),
            pltpu.VMEM((2 * cd,), jnp.float32),
            pltpu.SemaphoreType.DMA,
            pltpu.SemaphoreType.DMA,
            pltpu.SemaphoreType.DMA,
            pltpu.SemaphoreType.DMA,
        ],
    )
    def k(table_hbm, ids_hbm, out_hbm, tbl_v, ids_v, obuf_v, si0, si1, so0, so1):
        wid = lax.axis_index("s") * _NC + lax.axis_index("c")
        tok0 = wid * per_w
        pltpu.sync_copy(table_hbm, tbl_v)
        pltpu.async_copy(ids_hbm.at[pl.ds(tok0, chunk)],
                         ids_v.at[pl.ds(0, chunk)], si0)
        pltpu.async_copy(ids_hbm.at[pl.ds(tok0 + chunk, chunk)],
                         ids_v.at[pl.ds(chunk, chunk)], si1)

        lane = lax.iota(jnp.int32, _L)
        wout0 = lane * d

        def step(c, carry):
            p = lax.rem(c, 2)
            ioff = p * chunk
            ooff = p * cd

            @pl.when(p == 0)
            def _():
                pltpu.make_async_copy(ids_hbm.at[pl.ds(tok0, chunk)],
                                      ids_v.at[pl.ds(0, chunk)], si0).wait()

            @pl.when(p == 1)
            def _():
                pltpu.make_async_copy(ids_hbm.at[pl.ds(tok0, chunk)],
                                      ids_v.at[pl.ds(0, chunk)], si1).wait()

            @pl.when(jnp.logical_and(c >= 2, p == 0))
            def _():
                pltpu.make_async_copy(obuf_v.at[pl.ds(0, cd)],
                                      out_hbm.at[pl.ds(0, cd)], so0).wait()

            @pl.when(jnp.logical_and(c >= 2, p == 1))
            def _():
                pltpu.make_async_copy(obuf_v.at[pl.ds(0, cd)],
                                      out_hbm.at[pl.ds(0, cd)], so1).wait()

            @plsc.parallel_loop(0, chunk // _L, unroll=1)
            def _(blk):
                ids16 = ids_v[pl.ds(ioff + blk * _L, _L)]
                obase0 = ooff + blk * (_L * d)
                for l in range(_L):
                    rbase = ids16[l] * d
                    obase = obase0 + l * d
                    for jj in range(0, d, _L):
                        obuf_v[pl.ds(obase + jj, _L)] = (
                            tbl_v[pl.ds(rbase + jj, _L)])

            dst = out_hbm.at[pl.ds((tok0 + c * chunk) * d, cd)]

            @pl.when(p == 0)
            def _():
                pltpu.async_copy(obuf_v.at[pl.ds(0, cd)], dst, so0)

            @pl.when(p == 1)
            def _():
                pltpu.async_copy(obuf_v.at[pl.ds(cd, cd)], dst, so1)

            @pl.when(jnp.logical_and(c + 2 < n_chunks, p == 0))
            def _():
                pltpu.async_copy(
                    ids_hbm.at[pl.ds(tok0 + (c + 2) * chunk, chunk)],
                    ids_v.at[pl.ds(0, chunk)], si0)

            @pl.when(jnp.logical_and(c + 2 < n_chunks, p == 1))
            def _():
                pltpu.async_copy(
                    ids_hbm.at[pl.ds(tok0 + (c + 2) * chunk, chunk)],
                    ids_v.at[pl.ds(chunk, chunk)], si1)

            return carry

        lax.fori_loop(0, n_chunks, step, 0)

        pltpu.make_async_copy(obuf_v.at[pl.ds(0, cd)],
                              out_hbm.at[pl.ds(0, cd)], so0).wait()
        pltpu.make_async_copy(obuf_v.at[pl.ds(0, cd)],
                              out_hbm.at[pl.ds(0, cd)], so1).wait()

    return k(table_flat, ids_flat)


def kernel(token_ids, W, b):
    bsz, seq = token_ids.shape
    d = W.shape[1]
    n = bsz * seq
    table = _fused_table(W, b)
    out = _sc_expand(table.reshape(_TABLE_ROWS * d), token_ids.reshape(n), n, d)
    return out.reshape(bsz, seq, d)
